# baseline (speedup 1.0000x reference)
import jax
import jax.numpy as jnp
from jax import lax
from jax.experimental import pallas as pl
from jax.experimental.pallas import tpu as pltpu

_B = 4096
_NUM_CLASSES = 64
_ROWS = 256
_RC = 64
_NS = 32

_THRESH = 0.5
_MARGIN = 0.1
_SCALE_POS = 2.0
_SCALE_NEG = 40.0
_EPS = 1e-5
_LOG2E = 1.4426950408889634
_THR_CAP = 88.0


def _body(sim_ref, labr_ref, labc_ref, c_ref, out_ref, t_ref):
    part = jnp.zeros((1, 1), jnp.float32)
    for rc in range(_ROWS // _RC):
        r0 = rc * _RC
        lab_c = labc_ref[r0:r0 + _RC, :1]

        def p1(k, carry):
            mx, mnt, mxt = carry
            sk = sim_ref[pl.ds(r0, _RC), k, :]
            labk = labr_ref[0, k, :].reshape(1, 128)
            tk = jnp.where(lab_c == labk, sk - 2.0, sk)
            t_ref[pl.ds(r0, _RC), k, :] = tk
            return (jnp.maximum(mx, sk), jnp.minimum(mnt, tk),
                    jnp.maximum(mxt, tk))

        finit = jnp.full((_RC, 128), -3.0, jnp.float32)
        mx, mnt, mxt = lax.fori_loop(0, _NS, p1, (finit, -finit, finit))

        row_max = jnp.max(mx, axis=1, keepdims=True)
        min_pos = jnp.min(mnt, axis=1, keepdims=True) + 2.0
        max_t = jnp.max(mxt, axis=1, keepdims=True)
        max_neg = jnp.where(max_t >= -0.5, max_t, -jnp.inf)
        min_pos = jnp.where(min_pos < row_max - _EPS, min_pos, jnp.inf)

        pos_thr = jnp.minimum(row_max - _EPS, max_neg + _MARGIN)
        neg_thr = min_pos - _MARGIN
        u_pos_thr = jnp.minimum(-_SCALE_POS * (pos_thr - _THRESH), _THR_CAP)
        u_neg_thr = jnp.minimum(_SCALE_NEG * (neg_thr - _THRESH), _THR_CAP)

        a_pos = -_SCALE_POS * _LOG2E
        a_neg = _SCALE_NEG * _LOG2E
        b_pos = (_THRESH * _SCALE_POS - u_pos_thr) * _LOG2E + 2.0 * a_pos
        b_neg = (-_THRESH * _SCALE_NEG - u_neg_thr) * _LOG2E

        def p2(k, g):
            tk = t_ref[pl.ds(r0, _RC), k, :]
            same2 = tk < -0.5
            a = jnp.where(same2, a_pos, a_neg)
            b = jnp.where(same2, b_pos, b_neg)
            w = tk * a + b
            e = jnp.where(w > 0.0, jnp.exp2(w), 0.0)
            return g + jax.lax.dot_general(
                e, c_ref[k], (((1,), (0,)), ((), ())),
                preferred_element_type=jnp.float32)

        g = lax.fori_loop(0, _NS, p2, jnp.zeros((_RC, _NUM_CLASSES),
                                                jnp.float32))

        esum = jnp.sum(g, axis=1, keepdims=True)
        rowhot = lab_c == jax.lax.broadcasted_iota(
            jnp.int32, (1, _NUM_CLASSES), 1)
        psum_raw = jnp.sum(jnp.where(rowhot, g, 0.0), axis=1, keepdims=True)

        psum = psum_raw * jnp.exp(u_pos_thr)
        nsum = (esum - psum_raw) * jnp.exp(u_neg_thr)

        per_row = jnp.log1p(psum) / _SCALE_POS + jnp.log1p(nsum) / _SCALE_NEG
        valid = lab_c != 0
        part = part + jnp.sum(jnp.where(valid, per_row, 0.0),
                              axis=0, keepdims=True) * (1.0 / _B)

    @pl.when(pl.program_id(0) == 0)
    def _():
        out_ref[...] = jnp.zeros((1, 1), jnp.float32)

    out_ref[...] += part


def kernel(sim_mat, labels):
    sim3 = sim_mat.reshape(_B, _NS, 128)
    lab_r = labels.reshape(1, _NS, 128)
    lab_c = jnp.broadcast_to(labels.reshape(_B, 1), (_B, 128))
    c_mat = (labels.reshape(_B, 1)
             == jnp.arange(_NUM_CLASSES, dtype=jnp.int32).reshape(1, _NUM_CLASSES)
             ).astype(jnp.float32).reshape(_NS, 128, _NUM_CLASSES)
    out = pl.pallas_call(
        _body,
        grid=(_B // _ROWS,),
        in_specs=[
            pl.BlockSpec((_ROWS, _NS, 128), lambda i: (i, 0, 0)),
            pl.BlockSpec((1, _NS, 128), lambda i: (0, 0, 0)),
            pl.BlockSpec((_ROWS, 128), lambda i: (i, 0)),
            pl.BlockSpec((_NS, 128, _NUM_CLASSES), lambda i: (0, 0, 0)),
        ],
        out_specs=pl.BlockSpec((1, 1), lambda i: (0, 0)),
        out_shape=jax.ShapeDtypeStruct((1, 1), jnp.float32),
        scratch_shapes=[pltpu.VMEM((_ROWS, _NS, 128), jnp.float32)],
    )(sim3, lab_r, lab_c, c_mat)
    return out[0, 0]


# speedup vs baseline: 17.4981x; 17.4981x over previous
import jax
import jax.numpy as jnp
from jax.experimental import pallas as pl

_B = 4096
_NUM_CLASSES = 64
_ROWS = 256

_THRESH = 0.5
_MARGIN = 0.1
_SCALE_POS = 2.0
_SCALE_NEG = 40.0
_EPS = 1e-5
_LOG2E = 1.4426950408889634
_THR_CAP = 88.0


def _body(sim_ref, labr_ref, labc_ref, c_ref, out_ref):
    s = sim_ref[...]
    lab_r = labr_ref[...]
    lab_c = labc_ref[:, :1]
    same = lab_c == lab_r

    t = jnp.where(same, s - 2.0, s)
    row_max = jnp.max(s, axis=1, keepdims=True)
    min_pos = jnp.min(t, axis=1, keepdims=True) + 2.0
    max_t = jnp.max(t, axis=1, keepdims=True)
    max_neg = jnp.where(max_t >= -0.5, max_t, -jnp.inf)
    min_pos = jnp.where(min_pos < row_max - _EPS, min_pos, jnp.inf)

    pos_thr = jnp.minimum(row_max - _EPS, max_neg + _MARGIN)
    neg_thr = min_pos - _MARGIN

    u_pos_thr = jnp.minimum(-_SCALE_POS * (pos_thr - _THRESH), _THR_CAP)
    u_neg_thr = jnp.minimum(_SCALE_NEG * (neg_thr - _THRESH), _THR_CAP)

    a_pos = -_SCALE_POS * _LOG2E
    a_neg = _SCALE_NEG * _LOG2E
    b_pos = (_THRESH * _SCALE_POS - u_pos_thr) * _LOG2E
    b_neg = (-_THRESH * _SCALE_NEG - u_neg_thr) * _LOG2E
    same2 = t < -0.5
    a = jnp.where(same2, a_pos, a_neg)
    b = jnp.where(same2, b_pos + 2.0 * a_pos, b_neg)
    w = t * a + b
    e = jnp.where(w > 0.0, jnp.exp2(w), 0.0)

    g = jax.lax.dot_general(e, c_ref[...], (((1,), (0,)), ((), ())),
                            preferred_element_type=jnp.float32)
    esum = jnp.sum(g, axis=1, keepdims=True)
    rowhot = lab_c == jax.lax.broadcasted_iota(jnp.int32, (1, _NUM_CLASSES), 1)
    psum_raw = jnp.sum(jnp.where(rowhot, g, 0.0), axis=1, keepdims=True)

    psum = psum_raw * jnp.exp(u_pos_thr)
    nsum = (esum - psum_raw) * jnp.exp(u_neg_thr)

    per_row = jnp.log1p(psum) / _SCALE_POS + jnp.log1p(nsum) / _SCALE_NEG
    valid = lab_c != 0
    part = jnp.sum(jnp.where(valid, per_row, 0.0), axis=0, keepdims=True) * (1.0 / _B)

    @pl.when(pl.program_id(0) == 0)
    def _():
        out_ref[...] = jnp.zeros((1, 1), jnp.float32)

    out_ref[...] += part


def kernel(sim_mat, labels):
    lab_r = labels.reshape(1, _B)
    lab_c = jnp.broadcast_to(labels.reshape(_B, 1), (_B, 128))
    c_mat = (labels.reshape(_B, 1)
             == jnp.arange(_NUM_CLASSES, dtype=jnp.int32).reshape(1, _NUM_CLASSES)
             ).astype(jnp.float32)
    out = pl.pallas_call(
        _body,
        grid=(_B // _ROWS,),
        in_specs=[
            pl.BlockSpec((_ROWS, _B), lambda i: (i, 0)),
            pl.BlockSpec((1, _B), lambda i: (0, 0)),
            pl.BlockSpec((_ROWS, 128), lambda i: (i, 0)),
            pl.BlockSpec((_B, _NUM_CLASSES), lambda i: (0, 0)),
        ],
        out_specs=pl.BlockSpec((1, 1), lambda i: (0, 0)),
        out_shape=jax.ShapeDtypeStruct((1, 1), jnp.float32),
    )(sim_mat, lab_r, lab_c, c_mat)
    return out[0, 0]
